# Initial kernel scaffold; baseline (speedup 1.0000x reference)
import jax
import jax.numpy as jnp
from jax.experimental import pallas as pl


def kernel(keys, table):
    raise NotImplementedError("write your pallas kernel here")


# speedup vs baseline: 1.5610x; 1.5610x over previous
import functools

import jax
import jax.numpy as jnp
from jax import lax
from jax.experimental import pallas as pl
from jax.experimental.pallas import tpu as pltpu
from jax.experimental.pallas import tpu_sc as plsc

EMB_DIM = 32

_info = plsc.get_sparse_core_info()
_NC, _NS = _info.num_cores, _info.num_subcores
_NW = _NC * _NS

_CHUNK = 128
_K = 8


@functools.cache
def _make_gather(B: int):
    b_per_w = B // _NW
    nchunk = b_per_w // _CHUNK
    ngroup = nchunk // _K
    assert B % _NW == 0 and b_per_w % _CHUNK == 0 and nchunk % _K == 0

    mesh = plsc.VectorSubcoreMesh(core_axis_name="c", subcore_axis_name="s")

    @functools.partial(
        pl.kernel,
        mesh=mesh,
        out_type=jax.ShapeDtypeStruct((B, EMB_DIM), jnp.float32),
        scratch_types=[
            pltpu.VMEM((nchunk, _CHUNK), jnp.int32),
            pltpu.VMEM((_K, _CHUNK, EMB_DIM), jnp.float32),
            pltpu.SemaphoreType.DMA((_K,)),
        ],
        compiler_params=pltpu.CompilerParams(use_tc_tiling_on_sc=False),
    )
    def gather_kernel(keys_hbm, table_hbm, out_hbm, idx_v, rows_v, gsem):
        wid = lax.axis_index("s") * _NC + lax.axis_index("c")
        base = wid * b_per_w
        pltpu.sync_copy(keys_hbm.at[wid], idx_v)

        def group(g, carry):
            copies = []
            for b in range(_K):
                c = g * _K + b
                copies.append(
                    pltpu.async_copy(
                        table_hbm.at[idx_v.at[c]], rows_v.at[b], gsem.at[b]
                    )
                )
            for b in range(_K):
                c = g * _K + b
                copies[b].wait()
                pltpu.sync_copy(
                    rows_v.at[b],
                    out_hbm.at[pl.ds(base + c * _CHUNK, _CHUNK)],
                )
            return carry

        lax.fori_loop(0, ngroup, group, 0)

    return gather_kernel


def kernel(keys, table):
    batch, fields = keys.shape
    B = batch * fields
    b_per_w = B // _NW
    nchunk = b_per_w // _CHUNK
    karr = keys.reshape(-1).astype(jnp.int32).reshape(_NW, nchunk, _CHUNK)
    out = _make_gather(B)(karr, table)
    return out.reshape(batch, fields, EMB_DIM)


# speedup vs baseline: 1.5712x; 1.0066x over previous
import functools

import jax
import jax.numpy as jnp
from jax import lax
from jax.experimental import pallas as pl
from jax.experimental.pallas import tpu as pltpu
from jax.experimental.pallas import tpu_sc as plsc

EMB_DIM = 32

_info = plsc.get_sparse_core_info()
_NC, _NS = _info.num_cores, _info.num_subcores
_NW = _NC * _NS

_CHUNK = 128
_K = 26


@functools.cache
def _make_gather(B: int):
    b_per_w = B // _NW
    nchunk = b_per_w // _CHUNK
    ngroup = nchunk // _K
    assert B % _NW == 0 and b_per_w % _CHUNK == 0 and nchunk % _K == 0

    mesh = plsc.VectorSubcoreMesh(core_axis_name="c", subcore_axis_name="s")

    @functools.partial(
        pl.kernel,
        mesh=mesh,
        out_type=jax.ShapeDtypeStruct((B, EMB_DIM), jnp.float32),
        scratch_types=[
            pltpu.VMEM((nchunk, _CHUNK), jnp.int32),
            pltpu.VMEM((_K, _CHUNK, EMB_DIM), jnp.float32),
            pltpu.SemaphoreType.DMA((_K,)),
        ],
        compiler_params=pltpu.CompilerParams(use_tc_tiling_on_sc=False),
    )
    def gather_kernel(keys_hbm, table_hbm, out_hbm, idx_v, rows_v, gsem):
        wid = lax.axis_index("s") * _NC + lax.axis_index("c")
        base = wid * b_per_w
        pltpu.sync_copy(keys_hbm.at[wid], idx_v)

        def group(g, carry):
            copies = []
            for b in range(_K):
                c = g * _K + b
                copies.append(
                    pltpu.async_copy(
                        table_hbm.at[idx_v.at[c]], rows_v.at[b], gsem.at[b]
                    )
                )
            for b in range(_K):
                c = g * _K + b
                copies[b].wait()
                pltpu.sync_copy(
                    rows_v.at[b],
                    out_hbm.at[pl.ds(base + c * _CHUNK, _CHUNK)],
                )
            return carry

        lax.fori_loop(0, ngroup, group, 0)

    return gather_kernel


def kernel(keys, table):
    batch, fields = keys.shape
    B = batch * fields
    b_per_w = B // _NW
    nchunk = b_per_w // _CHUNK
    karr = keys.reshape(-1).astype(jnp.int32).reshape(_NW, nchunk, _CHUNK)
    out = _make_gather(B)(karr, table)
    return out.reshape(batch, fields, EMB_DIM)


# speedup vs baseline: 1.5730x; 1.0011x over previous
import functools

import jax
import jax.numpy as jnp
from jax import lax
from jax.experimental import pallas as pl
from jax.experimental.pallas import tpu as pltpu
from jax.experimental.pallas import tpu_sc as plsc

EMB_DIM = 32

_info = plsc.get_sparse_core_info()
_NC, _NS = _info.num_cores, _info.num_subcores
_NW = _NC * _NS

_CHUNK = 256
_K = 13


@functools.cache
def _make_gather(B: int):
    b_per_w = B // _NW
    nchunk = b_per_w // _CHUNK
    ngroup = nchunk // _K
    assert B % _NW == 0 and b_per_w % _CHUNK == 0 and nchunk % _K == 0

    mesh = plsc.VectorSubcoreMesh(core_axis_name="c", subcore_axis_name="s")

    @functools.partial(
        pl.kernel,
        mesh=mesh,
        out_type=jax.ShapeDtypeStruct((B, EMB_DIM), jnp.float32),
        scratch_types=[
            pltpu.VMEM((nchunk, _CHUNK), jnp.int32),
            pltpu.VMEM((_K, _CHUNK, EMB_DIM), jnp.float32),
            pltpu.SemaphoreType.DMA((_K,)),
        ],
        compiler_params=pltpu.CompilerParams(use_tc_tiling_on_sc=False),
    )
    def gather_kernel(keys_hbm, table_hbm, out_hbm, idx_v, rows_v, gsem):
        wid = lax.axis_index("s") * _NC + lax.axis_index("c")
        base = wid * b_per_w
        pltpu.sync_copy(keys_hbm.at[wid], idx_v)

        def group(g, carry):
            copies = []
            for b in range(_K):
                c = g * _K + b
                copies.append(
                    pltpu.async_copy(
                        table_hbm.at[idx_v.at[c]], rows_v.at[b], gsem.at[b]
                    )
                )
            for b in range(_K):
                c = g * _K + b
                copies[b].wait()
                pltpu.sync_copy(
                    rows_v.at[b],
                    out_hbm.at[pl.ds(base + c * _CHUNK, _CHUNK)],
                )
            return carry

        lax.fori_loop(0, ngroup, group, 0)

    return gather_kernel


def kernel(keys, table):
    batch, fields = keys.shape
    B = batch * fields
    b_per_w = B // _NW
    nchunk = b_per_w // _CHUNK
    karr = keys.reshape(-1).astype(jnp.int32).reshape(_NW, nchunk, _CHUNK)
    out = _make_gather(B)(karr, table)
    return out.reshape(batch, fields, EMB_DIM)
